# baseline (speedup 1.0000x reference)
import functools

import jax
import jax.numpy as jnp
from jax import lax
from jax.experimental import pallas as pl
from jax.experimental.pallas import tpu as pltpu
from jax.experimental.pallas import tpu_sc as plsc

N_FIELDS = 13
FSPLIT = 6
VOCAB = 100000
VPAD = 100096
DIM = 16
BATCH = 16384

NC = 2
NS = 16
NW = NC * NS
SPW = BATCH // NW
C = 128
NCH = SPW // C
NG = C // 16


NF2 = N_FIELDS - FSPLIT


def _user1_kernel(q, ix, dummy, qaccT, ix_v, rq, qa_v, sem):
    wid = lax.axis_index("s") * NC + lax.axis_index("c")
    base = wid * SPW
    pltpu.sync_copy(ix.at[:, pl.ds(base, SPW)], ix_v)

    def chunk_body(c, carry):
        co = c * C

        def fire(f, carry2):
            ii = ix_v.at[f, pl.ds(co, C)]
            for d in range(DIM):
                pltpu.async_copy(q.at[f * DIM + d].at[ii], rq.at[f * DIM + d], sem)
            return carry2

        lax.fori_loop(0, FSPLIT, fire, 0)
        pltpu.make_async_copy(dummy.at[pl.ds(0, FSPLIT * DIM)], rq, sem).wait()

        def gbody(g, carry3):
            o = g * 16
            for d in range(DIM):
                v = rq[d, pl.ds(o, 16)]
                for f in range(1, FSPLIT):
                    v = v + rq[f * DIM + d, pl.ds(o, 16)]
                qa_v[d, pl.ds(co + o, 16)] = v
            return carry3

        lax.fori_loop(0, NG, gbody, 0)
        return carry

    lax.fori_loop(0, NCH, chunk_body, 0)
    pltpu.sync_copy(qa_v, qaccT.at[:, pl.ds(base, SPW)])


def _user2_kernel(q, b, ix, prevT, dummy, qaccT, bacc,
                  ix_v, rq, rb, qa_v, ba_v, sem):
    wid = lax.axis_index("s") * NC + lax.axis_index("c")
    base = wid * SPW
    pltpu.sync_copy(ix.at[:, pl.ds(base, SPW)], ix_v)
    pltpu.sync_copy(prevT.at[:, pl.ds(base, SPW)], qa_v)

    def chunk_body(c, carry):
        co = c * C

        def fire(f, carry2):
            ii = ix_v.at[FSPLIT + f, pl.ds(co, C)]
            for d in range(DIM):
                pltpu.async_copy(q.at[f * DIM + d].at[ii], rq.at[f * DIM + d], sem)
            return carry2

        lax.fori_loop(0, NF2, fire, 0)

        def fireb(f, carry2):
            ii = ix_v.at[f, pl.ds(co, C)]
            pltpu.async_copy(b.at[f].at[ii], rb.at[f], sem)
            return carry2

        lax.fori_loop(0, N_FIELDS, fireb, 0)
        pltpu.make_async_copy(dummy.at[pl.ds(0, NF2 * DIM)], rq, sem).wait()
        pltpu.make_async_copy(dummy.at[pl.ds(0, N_FIELDS)], rb, sem).wait()

        def gbody(g, carry3):
            o = g * 16
            for d in range(DIM):
                v = rq[d, pl.ds(o, 16)]
                for f in range(1, NF2):
                    v = v + rq[f * DIM + d, pl.ds(o, 16)]
                qa_v[d, pl.ds(co + o, 16)] = qa_v[d, pl.ds(co + o, 16)] + v
            bv = rb[0, pl.ds(o, 16)]
            for f in range(1, N_FIELDS):
                bv = bv + rb[f, pl.ds(o, 16)]
            ba_v[pl.ds(co + o, 16)] = bv
            return carry3

        lax.fori_loop(0, NG, gbody, 0)
        return carry

    lax.fori_loop(0, NCH, chunk_body, 0)
    pltpu.sync_copy(qa_v, qaccT.at[:, pl.ds(base, SPW)])
    pltpu.sync_copy(ba_v, bacc.at[pl.ds(base, SPW)])


def _item_kernel(qi, bi, ixi, qaccT, bua, dummy, out,
                 ixi_v, ri, bri, qa_v, ba_v, outv, sem):
    wid = lax.axis_index("s") * NC + lax.axis_index("c")
    base = wid * SPW
    pltpu.sync_copy(ixi.at[:, pl.ds(base, SPW)], ixi_v)
    pltpu.sync_copy(qaccT.at[:, pl.ds(base, SPW)], qa_v)
    pltpu.sync_copy(bua.at[pl.ds(base, SPW)], ba_v)

    def chunk_body(c, carry):
        co = c * C

        def fire(f, carry2):
            ii = ixi_v.at[f, pl.ds(co, C)]
            for d in range(DIM):
                pltpu.async_copy(qi.at[f * DIM + d].at[ii], ri.at[f * DIM + d], sem)
            pltpu.async_copy(bi.at[f].at[ii], bri.at[f], sem)
            return carry2

        lax.fori_loop(0, N_FIELDS, fire, 0)
        pltpu.make_async_copy(dummy, ri, sem).wait()
        pltpu.make_async_copy(dummy.at[pl.ds(0, N_FIELDS)], bri, sem).wait()

        def gbody(g, carry3):
            o = g * 16
            acc = ba_v[pl.ds(co + o, 16)]
            for f in range(N_FIELDS):
                acc = acc + bri[f, pl.ds(o, 16)]
            for d in range(DIM):
                qiv = ri[d, pl.ds(o, 16)]
                for f in range(1, N_FIELDS):
                    qiv = qiv + ri[f * DIM + d, pl.ds(o, 16)]
                acc = acc + qa_v[d, pl.ds(co + o, 16)] * qiv
            outv[pl.ds(co + o, 16)] = acc
            return carry3

        lax.fori_loop(0, NG, gbody, 0)
        return carry

    lax.fori_loop(0, NCH, chunk_body, 0)
    pltpu.sync_copy(outv, out.at[pl.ds(base, SPW)])


@functools.partial(jax.jit, static_argnames=())
def kernel(Q_user, B_user, Q_item, B_item, x_user, x_item):
    quA = Q_user[:FSPLIT].transpose(0, 2, 1).reshape(FSPLIT * DIM, VOCAB)
    quB = Q_user[FSPLIT:].transpose(0, 2, 1).reshape(NF2 * DIM, VOCAB)
    qi = Q_item.transpose(0, 2, 1).reshape(N_FIELDS * DIM, VOCAB)
    bu = jnp.pad(B_user.reshape(N_FIELDS, VOCAB), ((0, 0), (0, VPAD - VOCAB)))
    bi = jnp.pad(B_item.reshape(N_FIELDS, VOCAB), ((0, 0), (0, VPAD - VOCAB)))
    ixu = x_user.T
    ixi = x_item.T
    ixuA = ixu[:FSPLIT]
    dummy = jnp.zeros((N_FIELDS * DIM, C), jnp.float32)

    mesh = plsc.VectorSubcoreMesh(core_axis_name="c", subcore_axis_name="s")
    cp = pltpu.CompilerParams(use_tc_tiling_on_sc=False)

    run_u1 = pl.kernel(
        _user1_kernel,
        out_type=jax.ShapeDtypeStruct((DIM, BATCH), jnp.float32),
        mesh=mesh,
        compiler_params=cp,
        scratch_types=[
            pltpu.VMEM((FSPLIT, SPW), jnp.int32),
            pltpu.VMEM((FSPLIT * DIM, C), jnp.float32),
            pltpu.VMEM((DIM, SPW), jnp.float32),
            pltpu.SemaphoreType.DMA,
        ],
    )
    qaccT1 = run_u1(quA, ixuA, dummy)

    run_u2 = pl.kernel(
        _user2_kernel,
        out_type=(
            jax.ShapeDtypeStruct((DIM, BATCH), jnp.float32),
            jax.ShapeDtypeStruct((BATCH,), jnp.float32),
        ),
        mesh=mesh,
        compiler_params=cp,
        scratch_types=[
            pltpu.VMEM((N_FIELDS, SPW), jnp.int32),
            pltpu.VMEM((NF2 * DIM, C), jnp.float32),
            pltpu.VMEM((N_FIELDS, C), jnp.float32),
            pltpu.VMEM((DIM, SPW), jnp.float32),
            pltpu.VMEM((SPW,), jnp.float32),
            pltpu.SemaphoreType.DMA,
        ],
    )
    qaccT, bua = run_u2(quB, bu, ixu, qaccT1, dummy)

    run_item = pl.kernel(
        _item_kernel,
        out_type=jax.ShapeDtypeStruct((BATCH,), jnp.float32),
        mesh=mesh,
        compiler_params=cp,
        scratch_types=[
            pltpu.VMEM((N_FIELDS, SPW), jnp.int32),
            pltpu.VMEM((N_FIELDS * DIM, C), jnp.float32),
            pltpu.VMEM((N_FIELDS, C), jnp.float32),
            pltpu.VMEM((DIM, SPW), jnp.float32),
            pltpu.VMEM((SPW,), jnp.float32),
            pltpu.VMEM((SPW,), jnp.float32),
            pltpu.SemaphoreType.DMA,
        ],
    )
    r = run_item(qi, bi, ixi, qaccT, bua, dummy)
    return r.reshape(BATCH, 1)


# speedup vs baseline: 1.0850x; 1.0850x over previous
import functools

import jax
import jax.numpy as jnp
from jax import lax
from jax.experimental import pallas as pl
from jax.experimental.pallas import tpu as pltpu
from jax.experimental.pallas import tpu_sc as plsc

N_FIELDS = 13
VOCAB = 100000
VPAD = 100096
DIM = 16
BATCH = 16384

NC = 2
NS = 16
NW = NC * NS
SPW = BATCH // NW
C = 128
NCH = SPW // C
NG = C // 16
ROWS = N_FIELDS * DIM


def _user_kernel(qu, bu, ixu, dummy, qaccT, bua,
                 ixu_v, ru0, ru1, bru0, bru1, qa_v, ba_v, sem0, sem1):
    wid = lax.axis_index("s") * NC + lax.axis_index("c")
    base = wid * SPW
    pltpu.sync_copy(ixu.at[:, pl.ds(base, SPW)], ixu_v)
    rus, brus, sems = (ru0, ru1), (bru0, bru1), (sem0, sem1)

    def fire(c, p):
        co = c * C

        def fire_f(f, carry):
            iu = ixu_v.at[f, pl.ds(co, C)]
            for d in range(DIM):
                r = f * DIM + d
                pltpu.async_copy(qu.at[r].at[iu], rus[p].at[r], sems[p])
            pltpu.async_copy(bu.at[f].at[iu], brus[p].at[f], sems[p])
            return carry

        lax.fori_loop(0, N_FIELDS, fire_f, 0)

    fire(0, 0)
    for c in range(NCH):
        p = c % 2
        if c + 1 < NCH:
            fire(c + 1, (c + 1) % 2)
        pltpu.make_async_copy(dummy, rus[p], sems[p]).wait()
        pltpu.make_async_copy(dummy.at[pl.ds(0, N_FIELDS)], brus[p], sems[p]).wait()
        co = c * C
        ru, bru = rus[p], brus[p]

        def gbody(g, carry, ru=ru, bru=bru, co=co):
            o = g * 16
            for d in range(DIM):
                quv = ru[d, pl.ds(o, 16)]
                for f in range(1, N_FIELDS):
                    quv = quv + ru[f * DIM + d, pl.ds(o, 16)]
                qa_v[d, pl.ds(co + o, 16)] = quv
            bacc = bru[0, pl.ds(o, 16)]
            for f in range(1, N_FIELDS):
                bacc = bacc + bru[f, pl.ds(o, 16)]
            ba_v[pl.ds(co + o, 16)] = bacc
            return carry

        lax.fori_loop(0, NG, gbody, 0)

    pltpu.sync_copy(qa_v, qaccT.at[:, pl.ds(base, SPW)])
    pltpu.sync_copy(ba_v, bua.at[pl.ds(base, SPW)])


def _item_kernel(qi, bi, ixi, qaccT, bua, dummy, out,
                 ixi_v, ri0, ri1, bri0, bri1, qa_v, ba_v, outv, sem0, sem1):
    wid = lax.axis_index("s") * NC + lax.axis_index("c")
    base = wid * SPW
    pltpu.sync_copy(ixi.at[:, pl.ds(base, SPW)], ixi_v)
    pltpu.sync_copy(qaccT.at[:, pl.ds(base, SPW)], qa_v)
    pltpu.sync_copy(bua.at[pl.ds(base, SPW)], ba_v)
    ris, bris, sems = (ri0, ri1), (bri0, bri1), (sem0, sem1)

    def fire(c, p):
        co = c * C

        def fire_f(f, carry):
            ii = ixi_v.at[f, pl.ds(co, C)]
            for d in range(DIM):
                r = f * DIM + d
                pltpu.async_copy(qi.at[r].at[ii], ris[p].at[r], sems[p])
            pltpu.async_copy(bi.at[f].at[ii], bris[p].at[f], sems[p])
            return carry

        lax.fori_loop(0, N_FIELDS, fire_f, 0)

    fire(0, 0)
    for c in range(NCH):
        p = c % 2
        if c + 1 < NCH:
            fire(c + 1, (c + 1) % 2)
        pltpu.make_async_copy(dummy, ris[p], sems[p]).wait()
        pltpu.make_async_copy(dummy.at[pl.ds(0, N_FIELDS)], bris[p], sems[p]).wait()
        co = c * C
        ri, bri = ris[p], bris[p]

        def gbody(g, carry, ri=ri, bri=bri, co=co):
            o = g * 16
            acc = ba_v[pl.ds(co + o, 16)]
            for f in range(N_FIELDS):
                acc = acc + bri[f, pl.ds(o, 16)]
            for d in range(DIM):
                qiv = ri[d, pl.ds(o, 16)]
                for f in range(1, N_FIELDS):
                    qiv = qiv + ri[f * DIM + d, pl.ds(o, 16)]
                acc = acc + qa_v[d, pl.ds(co + o, 16)] * qiv
            outv[pl.ds(co + o, 16)] = acc
            return carry

        lax.fori_loop(0, NG, gbody, 0)

    pltpu.sync_copy(outv, out.at[pl.ds(base, SPW)])


@functools.partial(jax.jit, static_argnames=())
def kernel(Q_user, B_user, Q_item, B_item, x_user, x_item):
    qu = Q_user.transpose(0, 2, 1).reshape(ROWS, VOCAB)
    qi = Q_item.transpose(0, 2, 1).reshape(ROWS, VOCAB)
    bu = jnp.pad(B_user.reshape(N_FIELDS, VOCAB), ((0, 0), (0, VPAD - VOCAB)))
    bi = jnp.pad(B_item.reshape(N_FIELDS, VOCAB), ((0, 0), (0, VPAD - VOCAB)))
    ixu = x_user.T
    ixi = x_item.T
    dummy = jnp.zeros((ROWS, C), jnp.float32)

    mesh = plsc.VectorSubcoreMesh(core_axis_name="c", subcore_axis_name="s")
    cp = pltpu.CompilerParams(use_tc_tiling_on_sc=False)

    run_user = pl.kernel(
        _user_kernel,
        out_type=(
            jax.ShapeDtypeStruct((DIM, BATCH), jnp.float32),
            jax.ShapeDtypeStruct((BATCH,), jnp.float32),
        ),
        mesh=mesh,
        compiler_params=cp,
        scratch_types=[
            pltpu.VMEM((N_FIELDS, SPW), jnp.int32),
            pltpu.VMEM((ROWS, C), jnp.float32),
            pltpu.VMEM((ROWS, C), jnp.float32),
            pltpu.VMEM((N_FIELDS, C), jnp.float32),
            pltpu.VMEM((N_FIELDS, C), jnp.float32),
            pltpu.VMEM((DIM, SPW), jnp.float32),
            pltpu.VMEM((SPW,), jnp.float32),
            pltpu.SemaphoreType.DMA,
            pltpu.SemaphoreType.DMA,
        ],
    )
    qaccT, bua = run_user(qu, bu, ixu, dummy)

    run_item = pl.kernel(
        _item_kernel,
        out_type=jax.ShapeDtypeStruct((BATCH,), jnp.float32),
        mesh=mesh,
        compiler_params=cp,
        scratch_types=[
            pltpu.VMEM((N_FIELDS, SPW), jnp.int32),
            pltpu.VMEM((ROWS, C), jnp.float32),
            pltpu.VMEM((ROWS, C), jnp.float32),
            pltpu.VMEM((N_FIELDS, C), jnp.float32),
            pltpu.VMEM((N_FIELDS, C), jnp.float32),
            pltpu.VMEM((DIM, SPW), jnp.float32),
            pltpu.VMEM((SPW,), jnp.float32),
            pltpu.VMEM((SPW,), jnp.float32),
            pltpu.SemaphoreType.DMA,
            pltpu.SemaphoreType.DMA,
        ],
    )
    r = run_item(qi, bi, ixi, qaccT, bua, dummy)
    return r.reshape(BATCH, 1)
